# baseline (speedup 1.0000x reference)
import jax
import jax.numpy as jnp
from jax import lax
from jax.experimental import pallas as pl
from jax.experimental.pallas import tpu as pltpu
from jax.experimental.pallas import tpu_sc as plsc

B, N, C = 16, 16384, 3
NPOINTS = 2048
L = 16
HN = N // 2
STEPS = HN // L
UNROLL = 16
NS = 16


def _fps_body(coord_hbm, sampled_hbm, finit_hbm, out_hbm,
              x_ref, y_ref, z_ref, dist_ref, flag_ref, finit_ref,
              ebuf_ref, pbuf_ref, xch_ref):
    c = lax.axis_index("c")
    s = lax.axis_index("s")
    b = c * 8 + s // 2
    h = s % 2
    ps = s ^ 1
    hoff = h * HN

    base = b * N + hoff
    pltpu.sync_copy(coord_hbm.at[pl.ds((b * 3 + 0) * N, N)], x_ref)
    pltpu.sync_copy(coord_hbm.at[pl.ds((b * 3 + 1) * N, N)], y_ref)
    pltpu.sync_copy(coord_hbm.at[pl.ds((b * 3 + 2) * N, N)], z_ref)
    pltpu.sync_copy(sampled_hbm.at[pl.ds(base, HN)], flag_ref)
    pltpu.sync_copy(finit_hbm, finit_ref)

    lane = lax.iota(jnp.int32, L)
    lane0 = lane == 0
    one16 = jnp.full((L,), 1.0, jnp.float32)
    big = jnp.full((L,), 1e10, jnp.float32)
    i1 = jnp.full((L,), 1, jnp.int32)
    i2 = jnp.full((L,), 2, jnp.int32)

    @plsc.parallel_loop(0, STEPS, step=8)
    def _init(i):
        for u in range(8):
            dist_ref[pl.ds((i + u) * L, L)] = big

    bvec = jnp.full((L,), b, jnp.int32)
    f0 = plsc.load_gather(finit_ref, [bvec])
    mine0 = (f0 >= hoff) & (f0 < hoff + HN)
    plsc.store_scatter(flag_ref, [jnp.clip(f0 - hoff, 0, HN - 1)],
                       one16, mask=lane0 & mine0)

    def outer(t, fvec):
        cx = plsc.load_gather(x_ref, [fvec])
        cy = plsc.load_gather(y_ref, [fvec])
        cz = plsc.load_gather(z_ref, [fvec])

        rmax0 = tuple(jnp.full((L,), -1.0, jnp.float32) for _ in range(UNROLL))
        rstep0 = tuple(jnp.zeros((L,), jnp.int32) for _ in range(UNROLL))

        @plsc.parallel_loop(0, STEPS, step=UNROLL, carry=(rmax0, rstep0))
        def inner(j, carry):
            rmax, rstep = carry
            jbc = jnp.full((L,), j, jnp.int32)
            rmax_n, rstep_n = [], []
            for u in range(UNROLL):
                dsl = pl.ds((j + u) * L, L)
                csl = pl.ds(hoff + (j + u) * L, L)
                dx = x_ref[csl] - cx
                dy = y_ref[csl] - cy
                dz = z_ref[csl] - cz
                d = dx * dx + dy * dy + dz * dz
                dn = jnp.minimum(dist_ref[dsl], d)
                dist_ref[dsl] = dn
                m = dn > rmax[u]
                rmax_n.append(jnp.maximum(dn, rmax[u]))
                rstep_n.append(jnp.where(m, jbc, rstep[u]))
            return tuple(rmax_n), tuple(rstep_n)

        rmaxs, rsteps = inner
        acc_v = rmaxs[0]
        acc_i = rsteps[0] * L + lane
        for u in range(1, UNROLL):
            v = rmaxs[u]
            i = rsteps[u] * L + jnp.full((L,), u * L, jnp.int32) + lane
            take = (v > acc_v) | ((v == acc_v) & (i < acc_i))
            acc_v = jnp.where(take, v, acc_v)
            acc_i = jnp.where(take, i, acc_i)

        gmax = jnp.max(acc_v)
        cand = jnp.where(acc_v == gmax, acc_i, jnp.full((L,), 2 ** 30, jnp.int32))
        gidx = jnp.full((L,), jnp.min(cand), jnp.int32) + hoff
        gv = jnp.full((L,), gmax, jnp.float32)

        e = jnp.where(lane == 1, gv, plsc.bitcast(gidx, jnp.float32))
        par = t % 2 + 4
        ebuf_ref[...] = e
        pltpu.sync_copy(ebuf_ref, xch_ref.at[par, s])
        plsc.subcore_barrier()
        pltpu.sync_copy(xch_ref.at[par, ps], pbuf_ref)
        pval = plsc.load_gather(pbuf_ref, [i1])
        pidx = plsc.bitcast(plsc.load_gather(pbuf_ref, [i2]), jnp.int32)
        win = (gv > pval) | ((gv == pval) & (gidx < pidx))
        wg = jnp.where(win, gidx, pidx)
        plsc.store_scatter(flag_ref, [jnp.clip(wg - hoff, 0, HN - 1)],
                           one16, mask=lane0 & win)
        return wg

    lax.fori_loop(0, NPOINTS - 1, outer, f0)
    pltpu.sync_copy(flag_ref, out_hbm.at[pl.ds(base, HN)])


def kernel(coord, sampled, farthest_init):
    coord_t = jnp.transpose(coord, (0, 2, 1)).reshape(-1)
    finit = farthest_init.astype(jnp.int32)
    mesh = plsc.VectorSubcoreMesh(core_axis_name="c", subcore_axis_name="s")
    k = pl.kernel(
        _fps_body,
        mesh=mesh,
        compiler_params=pltpu.CompilerParams(needs_layout_passes=False),
        out_type=jax.ShapeDtypeStruct((B * N,), jnp.float32),
        scratch_types=[
            pltpu.VMEM((N,), jnp.float32),
            pltpu.VMEM((N,), jnp.float32),
            pltpu.VMEM((N,), jnp.float32),
            pltpu.VMEM((HN,), jnp.float32),
            pltpu.VMEM((HN,), jnp.float32),
            pltpu.VMEM((L,), jnp.int32),
            pltpu.VMEM((L,), jnp.float32),
            pltpu.VMEM((L,), jnp.float32),
            pltpu.VMEM_SHARED((8, NS, L), jnp.float32),
        ],
    )
    return k(coord_t, sampled, finit)


# speedup vs baseline: 1.0891x; 1.0891x over previous
import jax
import jax.numpy as jnp
from jax import lax
from jax.experimental import pallas as pl
from jax.experimental.pallas import tpu as pltpu
from jax.experimental.pallas import tpu_sc as plsc

B, N, C = 16, 16384, 3
NPOINTS = 2048
L = 16
HN = N // 2
STEPS = HN // L
UNROLL = 4
NS = 16


def _fps_body(coord_hbm, sampled_hbm, finit_hbm, out_hbm,
              x_ref, y_ref, z_ref, dist_ref, flag_ref, finit_ref,
              ebuf_ref, pbuf_ref, xch_ref):
    c = lax.axis_index("c")
    s = lax.axis_index("s")
    b = c * 8 + s // 2
    h = s % 2
    ps = s ^ 1
    hoff = h * HN

    base = b * N + hoff
    pltpu.sync_copy(coord_hbm.at[pl.ds((b * 3 + 0) * N, N)], x_ref)
    pltpu.sync_copy(coord_hbm.at[pl.ds((b * 3 + 1) * N, N)], y_ref)
    pltpu.sync_copy(coord_hbm.at[pl.ds((b * 3 + 2) * N, N)], z_ref)
    pltpu.sync_copy(sampled_hbm.at[pl.ds(base, HN)], flag_ref)
    pltpu.sync_copy(finit_hbm, finit_ref)

    lane = lax.iota(jnp.int32, L)
    lane0 = lane == 0
    one16 = jnp.full((L,), 1.0, jnp.float32)
    big = jnp.full((L,), 1e10, jnp.float32)
    i1 = jnp.full((L,), 1, jnp.int32)
    i2 = jnp.full((L,), 2, jnp.int32)

    @plsc.parallel_loop(0, STEPS, step=8)
    def _init(i):
        for u in range(8):
            dist_ref[pl.ds((i + u) * L, L)] = big

    bvec = jnp.full((L,), b, jnp.int32)
    f0 = plsc.load_gather(finit_ref, [bvec])
    mine0 = (f0 >= hoff) & (f0 < hoff + HN)
    plsc.store_scatter(flag_ref, [jnp.clip(f0 - hoff, 0, HN - 1)],
                       one16, mask=lane0 & mine0)

    def outer(t, fvec):
        cx = plsc.load_gather(x_ref, [fvec])
        cy = plsc.load_gather(y_ref, [fvec])
        cz = plsc.load_gather(z_ref, [fvec])

        rmax0 = tuple(jnp.full((L,), -1.0, jnp.float32) for _ in range(UNROLL))
        rstep0 = tuple(jnp.zeros((L,), jnp.int32) for _ in range(UNROLL))

        @plsc.parallel_loop(0, STEPS, step=UNROLL, carry=(rmax0, rstep0))
        def inner(j, carry):
            rmax, rstep = carry
            jbc = jnp.full((L,), j, jnp.int32)
            rmax_n, rstep_n = [], []
            for u in range(UNROLL):
                dsl = pl.ds((j + u) * L, L)
                csl = pl.ds(hoff + (j + u) * L, L)
                dx = x_ref[csl] - cx
                dy = y_ref[csl] - cy
                dz = z_ref[csl] - cz
                d = dx * dx + dy * dy + dz * dz
                dn = jnp.minimum(dist_ref[dsl], d)
                dist_ref[dsl] = dn
                m = dn > rmax[u]
                rmax_n.append(jnp.maximum(dn, rmax[u]))
                rstep_n.append(jnp.where(m, jbc, rstep[u]))
            return tuple(rmax_n), tuple(rstep_n)

        rmaxs, rsteps = inner
        acc_v = rmaxs[0]
        acc_i = rsteps[0] * L + lane
        for u in range(1, UNROLL):
            v = rmaxs[u]
            i = rsteps[u] * L + jnp.full((L,), u * L, jnp.int32) + lane
            take = (v > acc_v) | ((v == acc_v) & (i < acc_i))
            acc_v = jnp.where(take, v, acc_v)
            acc_i = jnp.where(take, i, acc_i)

        gmax = jnp.max(acc_v)
        cand = jnp.where(acc_v == gmax, acc_i, jnp.full((L,), 2 ** 30, jnp.int32))
        gidx = jnp.full((L,), jnp.min(cand), jnp.int32) + hoff
        gv = jnp.full((L,), gmax, jnp.float32)

        e = jnp.where(lane == 1, gv, plsc.bitcast(gidx, jnp.float32))
        par = t % 2 + 4
        ebuf_ref[...] = e
        pltpu.sync_copy(ebuf_ref, xch_ref.at[par, s])
        plsc.subcore_barrier()
        pltpu.sync_copy(xch_ref.at[par, ps], pbuf_ref)
        pval = plsc.load_gather(pbuf_ref, [i1])
        pidx = plsc.bitcast(plsc.load_gather(pbuf_ref, [i2]), jnp.int32)
        win = (gv > pval) | ((gv == pval) & (gidx < pidx))
        wg = jnp.where(win, gidx, pidx)
        plsc.store_scatter(flag_ref, [jnp.clip(wg - hoff, 0, HN - 1)],
                           one16, mask=lane0 & win)
        return wg

    lax.fori_loop(0, NPOINTS - 1, outer, f0)
    pltpu.sync_copy(flag_ref, out_hbm.at[pl.ds(base, HN)])


def kernel(coord, sampled, farthest_init):
    coord_t = jnp.transpose(coord, (0, 2, 1)).reshape(-1)
    finit = farthest_init.astype(jnp.int32)
    mesh = plsc.VectorSubcoreMesh(core_axis_name="c", subcore_axis_name="s")
    k = pl.kernel(
        _fps_body,
        mesh=mesh,
        compiler_params=pltpu.CompilerParams(needs_layout_passes=False),
        out_type=jax.ShapeDtypeStruct((B * N,), jnp.float32),
        scratch_types=[
            pltpu.VMEM((N,), jnp.float32),
            pltpu.VMEM((N,), jnp.float32),
            pltpu.VMEM((N,), jnp.float32),
            pltpu.VMEM((HN,), jnp.float32),
            pltpu.VMEM((HN,), jnp.float32),
            pltpu.VMEM((L,), jnp.int32),
            pltpu.VMEM((L,), jnp.float32),
            pltpu.VMEM((L,), jnp.float32),
            pltpu.VMEM_SHARED((8, NS, L), jnp.float32),
        ],
    )
    return k(coord_t, sampled, finit)
